# Initial kernel scaffold; baseline (speedup 1.0000x reference)
import jax
import jax.numpy as jnp
from jax.experimental import pallas as pl


def kernel(x, Wr, br, sgW, svW, soW, sob, egW, evW, eoW, eob):
    raise NotImplementedError("write your pallas kernel here")


# speedup vs baseline: 1.2400x; 1.2400x over previous
import functools

import jax
import jax.numpy as jnp
from jax import lax
from jax.experimental import pallas as pl
from jax.experimental.pallas import tpu as pltpu

_S, _D, _H, _O = 2048, 1024, 1024, 1024
_E, _K = 8, 2
_TILE = 128
_CR = _S * _K + _E * _TILE
_NT = _CR // _TILE
_RTS = 256


def _router_body(x_ref, wr_ref, br_ref, logits_ref, idx_ref, wn_ref):
    xt = x_ref[...]
    l = jnp.dot(xt, wr_ref[...], preferred_element_type=jnp.float32) + br_ref[...]
    logits_ref[...] = l
    m = jnp.max(l, axis=1, keepdims=True)
    e = jnp.exp(l - m)
    w = e / jnp.sum(e, axis=1, keepdims=True)
    iota = lax.broadcasted_iota(jnp.int32, w.shape, 1)
    w1 = jnp.max(w, axis=1, keepdims=True)
    i1 = jnp.min(jnp.where(w == w1, iota, _E), axis=1, keepdims=True)
    wm = jnp.where(iota == i1, -1.0, w)
    w2 = jnp.max(wm, axis=1, keepdims=True)
    i2 = jnp.min(jnp.where(wm == w2, iota, _E), axis=1, keepdims=True)
    s = w1 + w2
    idx_ref[...] = jnp.concatenate([i1, i2], axis=1)
    wn_ref[...] = jnp.concatenate([w1 / s, w2 / s], axis=1)


def _router(x2, Wr, br):
    return pl.pallas_call(
        _router_body,
        grid=(_S // _RTS,),
        in_specs=[
            pl.BlockSpec((_RTS, _D), lambda i: (i, 0)),
            pl.BlockSpec((_D, _E), lambda i: (0, 0)),
            pl.BlockSpec((1, _E), lambda i: (0, 0)),
        ],
        out_specs=[
            pl.BlockSpec((_RTS, _E), lambda i: (i, 0)),
            pl.BlockSpec((_RTS, _K), lambda i: (i, 0)),
            pl.BlockSpec((_RTS, _K), lambda i: (i, 0)),
        ],
        out_shape=[
            jax.ShapeDtypeStruct((_S, _E), jnp.float32),
            jax.ShapeDtypeStruct((_S, _K), jnp.int32),
            jax.ShapeDtypeStruct((_S, _K), jnp.float32),
        ],
    )(x2, Wr, br.reshape(1, _E))


def _grouped_ffn_body(tile_eid_ref, xs_ref, gw_ref, vw_ref, ow_ref, ob_ref,
                      ws_ref, ys_ref):
    del tile_eid_ref
    xt = xs_ref[...]
    g = jnp.dot(xt, gw_ref[0], preferred_element_type=jnp.float32)
    v = jnp.dot(xt, vw_ref[0], preferred_element_type=jnp.float32)
    h = (g * jax.nn.sigmoid(g)) * v
    y = jnp.dot(h, ow_ref[0], preferred_element_type=jnp.float32) + ob_ref[0]
    ys_ref[...] = y * ws_ref[...]


def _grouped_ffn(tile_eid, xs, egW, evW, eoW, eob, w_row):
    grid_spec = pltpu.PrefetchScalarGridSpec(
        num_scalar_prefetch=1,
        grid=(_NT,),
        in_specs=[
            pl.BlockSpec((_TILE, _D), lambda i, te: (i, 0)),
            pl.BlockSpec((1, _D, _H), lambda i, te: (te[i], 0, 0)),
            pl.BlockSpec((1, _D, _H), lambda i, te: (te[i], 0, 0)),
            pl.BlockSpec((1, _H, _O), lambda i, te: (te[i], 0, 0)),
            pl.BlockSpec((1, 1, _O), lambda i, te: (te[i], 0, 0)),
            pl.BlockSpec((_TILE, 1), lambda i, te: (i, 0)),
        ],
        out_specs=pl.BlockSpec((_TILE, _O), lambda i, te: (i, 0)),
    )
    return pl.pallas_call(
        _grouped_ffn_body,
        grid_spec=grid_spec,
        out_shape=jax.ShapeDtypeStruct((_CR, _O), jnp.float32),
    )(tile_eid, xs, egW, evW, eoW, eob.reshape(_E, 1, _O), w_row)


def _shared_ffn_body(x_ref, gw_ref, vw_ref, ow_ref, ob_ref, y_ref):
    xt = x_ref[...]
    g = jnp.dot(xt, gw_ref[...], preferred_element_type=jnp.float32)
    v = jnp.dot(xt, vw_ref[...], preferred_element_type=jnp.float32)
    h = (g * jax.nn.sigmoid(g)) * v
    y_ref[...] = jnp.dot(h, ow_ref[...], preferred_element_type=jnp.float32) + ob_ref[...]


def _shared_ffn(x2, sgW, svW, soW, sob):
    return pl.pallas_call(
        _shared_ffn_body,
        grid=(_S // _RTS,),
        in_specs=[
            pl.BlockSpec((_RTS, _D), lambda i: (i, 0)),
            pl.BlockSpec((_D, _H), lambda i: (0, 0)),
            pl.BlockSpec((_D, _H), lambda i: (0, 0)),
            pl.BlockSpec((_H, _O), lambda i: (0, 0)),
            pl.BlockSpec((1, _O), lambda i: (0, 0)),
        ],
        out_specs=pl.BlockSpec((_RTS, _O), lambda i: (i, 0)),
        out_shape=jax.ShapeDtypeStruct((_S, _O), jnp.float32),
    )(x2, sgW, svW, soW, sob.reshape(1, _O))


def kernel(x, Wr, br, sgW, svW, soW, sob, egW, evW, eoW, eob):
    x2 = x.reshape(_S, _D)
    logits, topk_idx, wn = _router(x2, Wr, br)

    eid = topk_idx.reshape(-1)
    onehot = (eid[:, None] == jnp.arange(_E)[None, :]).astype(jnp.int32)
    counts = jnp.sum(onehot, axis=0)
    rank = jnp.take_along_axis(jnp.cumsum(onehot, axis=0) - onehot,
                               eid[:, None], axis=1)[:, 0]
    padded_counts = ((counts + _TILE - 1) // _TILE) * _TILE
    ends = jnp.cumsum(padded_counts)
    padded_offsets = ends - padded_counts
    pos = padded_offsets[eid] + rank
    row_token = jnp.zeros((_CR,), jnp.int32).at[pos].set(
        jnp.arange(_S * _K, dtype=jnp.int32) // _K)
    w_row = jnp.zeros((_CR, 1), jnp.float32).at[pos, 0].set(wn.reshape(-1))
    tile_eid = jnp.minimum(
        jnp.searchsorted(ends, jnp.arange(_NT, dtype=jnp.int32) * _TILE,
                         side="right"),
        _E - 1).astype(jnp.int32)

    xs = x2[row_token]

    ys = _grouped_ffn(tile_eid, xs, egW, evW, eoW, eob, w_row)
    ysh = _shared_ffn(x2, sgW, svW, soW, sob)

    p = pos.reshape(_S, _K)
    out = ys[p[:, 0]] + ys[p[:, 1]] + ysh

    return (out.reshape(1, _S, _O),
            logits.reshape(1, _S, _E),
            topk_idx.reshape(1, _S, _K))
